# baseline (speedup 1.0000x reference)
import functools

import jax
import jax.numpy as jnp
import numpy as np
from jax import lax
from jax.experimental import pallas as pl
from jax.experimental.pallas import tpu as pltpu
from jax.experimental.pallas import tpu_sc as plsc

DIM = 128
NC = 2
NS = 16
NW = NC * NS
EB = 32
GRP = 4
CHUNK = 12
NCHUNK = 2


def _qv_body(hv_ref, at_ref, out_ref):
    hv = hv_ref[...]
    z = jnp.dot(hv, at_ref[...], preferred_element_type=jnp.float32)
    out_ref[:, :DIM] = jnp.exp(2.0 * z)
    out_ref[:, DIM:] = hv


def _compute_qv(h_v, At):
    n = h_v.shape[0]
    blk = 400
    return pl.pallas_call(
        _qv_body,
        grid=(n // blk,),
        in_specs=[pl.BlockSpec((blk, DIM), lambda i: (i, 0)),
                  pl.BlockSpec((DIM, DIM), lambda i: (0, 0))],
        out_specs=pl.BlockSpec((blk, 2 * DIM), lambda i: (i, 0)),
        out_shape=jax.ShapeDtypeStruct((n, 2 * DIM), jnp.float32),
    )(h_v, At)


def _p_half_split(epw):
    half = epw // 2
    return half + (-half) % 8


def _compute_p_chunk(h_p, Bt, c, ec):
    epw = ec // NW
    S = _p_half_split(epw)

    def body(hp_ref, bt_ref, out_ref):
        z = jnp.dot(hp_ref[...], bt_ref[...],
                    preferred_element_type=jnp.float32)
        e2 = jnp.exp(2.0 * z)
        lo = jnp.concatenate([e2[:, b * 32:b * 32 + 16] for b in range(4)],
                             axis=1)
        hi = jnp.concatenate(
            [e2[:, b * 32 + 16:b * 32 + 32] for b in range(4)], axis=1)

        def rtne16(x):
            u = jax.lax.bitcast_convert_type(x, jnp.uint32)
            return (u + 0x7FFF + ((u >> 16) & 1)) >> 16
        w = jax.lax.bitcast_convert_type(rtne16(lo) | (rtne16(hi) << 16),
                                         jnp.int32)
        upper = jnp.concatenate(
            [w[S:], jnp.zeros((2 * S - epw, 64), jnp.int32)], axis=0)
        out_ref[...] = jnp.concatenate([w[:S], upper], axis=1)

    return pl.pallas_call(
        body,
        grid=(NW,),
        in_specs=[pl.BlockSpec((epw, DIM), lambda i, c=c: (c * NW + i, 0)),
                  pl.BlockSpec((DIM, DIM), lambda i: (0, 0))],
        out_specs=pl.BlockSpec((S, DIM), lambda i: (i, 0)),
        out_shape=jax.ShapeDtypeStruct((NW * S, DIM), jnp.int32),
    )(h_p, Bt)


def _sum_body(p_ref, o_ref):
    o_ref[...] = p_ref[0] + p_ref[1]


def _sum_partials(partials):
    n = partials.shape[1]
    blk = 1000
    return pl.pallas_call(
        _sum_body,
        grid=(n // blk,),
        in_specs=[pl.BlockSpec((2, blk, DIM), lambda i: (0, i, 0))],
        out_specs=pl.BlockSpec((blk, DIM), lambda i: (i, 0)),
        out_shape=jax.ShapeDtypeStruct((n, DIM), jnp.float32),
    )(partials)


def _make_sc_chunk(n_tgt, e_chunk, cbase, first):
    epw = e_chunk // NW
    S = _p_half_split(epw)
    RB = EB // 2
    nb = (S // RB) - (S // RB) % GRP
    ngrp = nb // GRP
    tail_rows = S - nb * RB
    rpt = (n_tgt // NS) - (n_tgt // NS) % 8
    r_tail = n_tgt - rpt * NS
    lpw = n_tgt // NW - (n_tgt // NW) % 8
    lf_full = lpw // EB
    lf_rem = lpw - lf_full * EB
    lf_tail = n_tgt - lpw * NW

    mesh = plsc.VectorSubcoreMesh(core_axis_name="c", subcore_axis_name="s")
    part_ty = jax.ShapeDtypeStruct((NC, n_tgt, DIM), jnp.float32)
    out_ty = ((part_ty, jax.ShapeDtypeStruct((n_tgt, DIM), jnp.float32))
              if first else part_ty)
    scratch = [pltpu.VMEM((CHUNK * RB,), jnp.int32)] * 4
    scratch += [pltpu.VMEM((EB,), jnp.int32)] * GRP
    scratch += [pltpu.VMEM((EB,), jnp.int32)] * GRP
    scratch += [
        pltpu.VMEM((EB, 2 * DIM), jnp.float32),
        pltpu.VMEM((EB, 2 * DIM), jnp.float32),
        pltpu.VMEM((EB, DIM), jnp.float32),
        pltpu.VMEM((EB, DIM), jnp.float32),
        pltpu.VMEM((RB, DIM), jnp.int32),
        pltpu.VMEM((RB, DIM), jnp.int32),
        pltpu.VMEM((EB, DIM), jnp.float32),
        pltpu.VMEM((EB, DIM), jnp.float32),
        pltpu.VMEM((16,), jnp.int32),
        pltpu.VMEM((16,), jnp.int32),
        pltpu.VMEM_SHARED((n_tgt, DIM), jnp.float32),
    ] + [pltpu.SemaphoreType.DMA] * 9

    def run(qv_hbm, p_hbm, ht_hbm, src_hbm, dst_hbm, part_out, scr,
            lidx_hbm, hv_hbm, last_out, part_in):
        (srcc_lo, srcc_hi, dstc_lo, dstc_hi,
         ss0, ss1, ss2, ss3, ds0, ds1, ds2, ds3,
         qv0, qv1, t0, t1, p0, p1, o0, o1, stail, dtail, acc,
         sq0, sq1, st0, st1, sp0, sp1, ssc0, ssc1, sem_l) = scr
        ssc = (ss0, ss1, ss2, ss3)
        dsc = (ds0, ds1, ds2, ds3)
        qvb = (qv0, qv1)
        tbufs = (t0, t1)
        pbufs = (p0, p1)
        obufs = (o0, o1)
        sem_q = (sq0, sq1)
        sem_t = (st0, st1)
        sem_p = (sp0, sp1)
        sem_s = (ssc0, ssc1)

        cid = lax.axis_index("c")
        sid = lax.axis_index("s")
        wid = cid * NS + sid

        zero16 = jnp.zeros((16,), jnp.float32)
        r0 = sid * rpt

        if part_in is None:
            def zrow(i, _):
                for g in range(8):
                    o0[i, pl.ds(g * 16, 16)] = zero16
                return 0
            lax.fori_loop(0, EB, zrow, 0)
            zfull = rpt // EB
            zrem = rpt - zfull * EB
            for k in range(zfull):
                pltpu.sync_copy(o0, acc.at[pl.ds(r0 + k * EB, EB)])
            if zrem:
                pltpu.sync_copy(o0.at[pl.ds(0, zrem)],
                                acc.at[pl.ds(r0 + zfull * EB, zrem)])
            if r_tail:
                @pl.when(sid == 0)
                def _():
                    pltpu.sync_copy(o0.at[pl.ds(0, r_tail)],
                                    acc.at[pl.ds(rpt * NS, r_tail)])
        else:
            pltpu.sync_copy(part_in.at[cid].at[pl.ds(r0, rpt)],
                            acc.at[pl.ds(r0, rpt)])
            if r_tail:
                @pl.when(sid == 0)
                def _():
                    pltpu.sync_copy(part_in.at[cid].at[pl.ds(rpt * NS, r_tail)],
                                    acc.at[pl.ds(rpt * NS, r_tail)])
        plsc.subcore_barrier()

        ibase = cbase + wid * epw
        pbase = wid * S

        def copy_idx_to_ring(slot, off):
            ssc[slot][pl.ds(0, 16)] = srcc_lo[pl.ds(off, 16)]
            ssc[slot][pl.ds(16, 16)] = srcc_hi[pl.ds(off, 16)]
            dsc[slot][pl.ds(0, 16)] = dstc_lo[pl.ds(off, 16)]
            dsc[slot][pl.ds(16, 16)] = dstc_hi[pl.ds(off, 16)]

        def fetch_idx_chunk(kstart):
            b_lo = ibase + kstart * RB
            b_hi = ibase + S + kstart * RB
            pltpu.sync_copy(src_hbm.at[pl.ds(b_lo, CHUNK * RB)], srcc_lo)
            pltpu.sync_copy(src_hbm.at[pl.ds(b_hi, CHUNK * RB)], srcc_hi)
            pltpu.sync_copy(dst_hbm.at[pl.ds(b_lo, CHUNK * RB)], dstc_lo)
            pltpu.sync_copy(dst_hbm.at[pl.ds(b_hi, CHUNK * RB)], dstc_hi)

        def issue_gathers(slot, par, prow):
            pltpu.async_copy(qv_hbm.at[ssc[slot]], qvb[par], sem_q[par])
            pltpu.async_copy(ht_hbm.at[dsc[slot]], tbufs[par], sem_t[par])
            pltpu.async_copy(p_hbm.at[pl.ds(prow, RB)], pbufs[par],
                             sem_p[par])

        def wait_gathers(slot, par, prow):
            pltpu.make_async_copy(qv_hbm.at[ssc[slot]], qvb[par],
                                  sem_q[par]).wait()
            pltpu.make_async_copy(ht_hbm.at[dsc[slot]], tbufs[par],
                                  sem_t[par]).wait()
            pltpu.make_async_copy(p_hbm.at[pl.ds(prow, RB)],
                                  pbufs[par], sem_p[par]).wait()

        def wait_scatter(par, slot):
            pltpu.make_async_copy(obufs[par], acc.at[dsc[slot]],
                                  sem_s[par]).wait()

        def compute_batch(par, half):
            qv = qvb[par]
            tt = tbufs[par]
            pp = pbufs[par]
            oo = obufs[par]

            @plsc.parallel_loop(0, half, unroll=2)
            def pair(r):
                for s in range(2):
                    i = s * half + r
                    s_acc = zero16
                    t_acc = zero16
                    for g in range(4):
                        w = pp[r, pl.ds(s * 64 + g * 16, 16)]
                        plo = plsc.bitcast(w << 16, jnp.float32)
                        phi = plsc.bitcast(w & jnp.int32(-65536), jnp.float32)
                        sl0 = pl.ds(g * 32, 16)
                        sl1 = pl.ds(g * 32 + 16, 16)
                        s_acc = s_acc + tt[i, sl0] / (plo * qv[i, sl0] + 1.0)
                        t_acc = t_acc + tt[i, sl0]
                        s_acc = s_acc + tt[i, sl1] / (phi * qv[i, sl1] + 1.0)
                        t_acc = t_acc + tt[i, sl1]
                    coef = jnp.sum(t_acc - 2.0 * s_acc)
                    for g in range(8):
                        oo[i, pl.ds(g * 16, 16)] = (
                            coef * qv[i, pl.ds(DIM + g * 16, 16)])

        fetch_idx_chunk(0)
        for b0 in (0, 1):
            copy_idx_to_ring(b0, b0 * RB)
            issue_gathers(b0, b0, pbase + b0 * RB)

        def group(g, _):
            for j in range(GRP):
                par = j % 2
                k = GRP * g + j
                wait_gathers(j, par, pbase + k * RB)
                if j < 2:
                    @pl.when(g > 0)
                    def _():
                        wait_scatter(par, (j + 2) % GRP)
                else:
                    wait_scatter(par, (j + 2) % GRP)
                compute_batch(par, RB)
                pltpu.async_copy(obufs[par], acc.at[dsc[j]], sem_s[par],
                                 add=True)
                if j == 2:
                    @pl.when((g % (CHUNK // GRP) == (CHUNK // GRP) - 1)
                             & (g < ngrp - 1))
                    def _():
                        fetch_idx_chunk(k + 2)
                nslot = (j + 2) % GRP

                def prefetch():
                    k2 = GRP * g + j + 2
                    off = (k2 % CHUNK) * RB
                    copy_idx_to_ring(nslot, off)
                    issue_gathers(nslot, par, pbase + k2 * RB)
                if j < 2:
                    prefetch()
                else:
                    @pl.when(g < ngrp - 1)
                    def _():
                        prefetch()
            return 0
        lax.fori_loop(0, ngrp, group, 0)
        wait_scatter(0, (nb - 2) % GRP)
        wait_scatter(1, (nb - 1) % GRP)

        if tail_rows:
            izero = jnp.zeros((16,), jnp.int32)
            stail[pl.ds(0, 16)] = izero
            dtail[pl.ds(0, 16)] = izero
            pltpu.sync_copy(src_hbm.at[pl.ds(ibase + nb * RB, tail_rows)],
                            stail.at[pl.ds(0, tail_rows)])
            pltpu.sync_copy(dst_hbm.at[pl.ds(ibase + nb * RB, tail_rows)],
                            dtail.at[pl.ds(0, tail_rows)])
            pltpu.async_copy(qv_hbm.at[stail], qv0.at[pl.ds(0, 16)],
                             sq0).wait()
            pltpu.async_copy(ht_hbm.at[dtail], t0.at[pl.ds(0, 16)],
                             st0).wait()
            pltpu.sync_copy(p_hbm.at[pl.ds(pbase + nb * RB, tail_rows)],
                            p0.at[pl.ds(0, tail_rows)])
            compute_batch(0, tail_rows)
            for i in range(tail_rows, 16):
                for g in range(8):
                    o0[i, pl.ds(g * 16, 16)] = zero16
            pltpu.sync_copy(o0.at[pl.ds(0, 16)], acc.at[dtail], add=True)

        if first:
            lbase = wid * lpw
            for r in range(lf_full):
                pltpu.sync_copy(lidx_hbm.at[pl.ds(lbase + r * EB, EB)], ss0)
                pltpu.async_copy(hv_hbm.at[ss0], t0, sem_l).wait()
                pltpu.sync_copy(t0, last_out.at[pl.ds(lbase + r * EB, EB)])
            if lf_rem:
                rb = lbase + lf_full * EB
                pltpu.sync_copy(lidx_hbm.at[pl.ds(rb, lf_rem)],
                                ss0.at[pl.ds(0, lf_rem)])
                pltpu.async_copy(hv_hbm.at[ss0.at[pl.ds(0, lf_rem)]],
                                 t0.at[pl.ds(0, lf_rem)], sem_l).wait()
                pltpu.sync_copy(t0.at[pl.ds(0, lf_rem)],
                                last_out.at[pl.ds(rb, lf_rem)])
            if lf_tail:
                @pl.when(wid == NW - 1)
                def _():
                    tb = lpw * NW
                    pltpu.sync_copy(lidx_hbm.at[pl.ds(tb, lf_tail)], stail)
                    pltpu.async_copy(hv_hbm.at[stail],
                                     t0.at[pl.ds(0, lf_tail)], sem_l).wait()
                    pltpu.sync_copy(t0.at[pl.ds(0, lf_tail)],
                                    last_out.at[pl.ds(tb, lf_tail)])

        plsc.subcore_barrier()
        pltpu.sync_copy(acc.at[pl.ds(r0, rpt)],
                        part_out.at[cid].at[pl.ds(r0, rpt)])
        if r_tail:
            @pl.when(sid == 0)
            def _():
                pltpu.sync_copy(acc.at[pl.ds(rpt * NS, r_tail)],
                                part_out.at[cid].at[pl.ds(rpt * NS, r_tail)])

    kw = dict(out_type=out_ty, mesh=mesh,
              compiler_params=pltpu.CompilerParams(needs_layout_passes=False),
              scratch_types=scratch)
    if first:
        @functools.partial(pl.kernel, **kw)
        def sc_first(qv_hbm, p_hbm, ht_hbm, src_hbm, dst_hbm, lidx_hbm,
                     hv_hbm, part_out, last_out,
                     sc1, sc2, sc3, sc4, ss0, ss1, ss2, ss3,
                     ds0, ds1, ds2, ds3,
                     qv0, qv1, t0, t1, p0, p1, o0, o1, stail, dtail, acc,
                     sq0, sq1, st0, st1, sp0, sp1, ssc0, ssc1, sem_l):
            scr = (sc1, sc2, sc3, sc4, ss0, ss1, ss2, ss3,
                   ds0, ds1, ds2, ds3,
                   qv0, qv1, t0, t1, p0, p1, o0, o1, stail, dtail, acc,
                   sq0, sq1, st0, st1, sp0, sp1, ssc0, ssc1, sem_l)
            run(qv_hbm, p_hbm, ht_hbm, src_hbm, dst_hbm, part_out, scr,
                lidx_hbm, hv_hbm, last_out, None)
        return sc_first
    else:
        @functools.partial(pl.kernel, **kw)
        def sc_rest(qv_hbm, p_hbm, ht_hbm, src_hbm, dst_hbm, part_in,
                    part_out,
                    sc1, sc2, sc3, sc4, ss0, ss1, ss2, ss3,
                    ds0, ds1, ds2, ds3,
                    qv0, qv1, t0, t1, p0, p1, o0, o1, stail, dtail, acc,
                    sq0, sq1, st0, st1, sp0, sp1, ssc0, ssc1, sem_l):
            scr = (sc1, sc2, sc3, sc4, ss0, ss1, ss2, ss3,
                   ds0, ds1, ds2, ds3,
                   qv0, qv1, t0, t1, p0, p1, o0, o1, stail, dtail, acc,
                   sq0, sq1, st0, st1, sp0, sp1, ssc0, ssc1, sem_l)
            run(qv_hbm, p_hbm, ht_hbm, src_hbm, dst_hbm, part_out, scr,
                None, None, None, part_in)
        return sc_rest


def kernel(h_v, h_p, h_t, edge_index, last_idx, W_q):
    n_tgt = h_t.shape[0]
    n_edge = h_p.shape[0]
    At = W_q[:, :DIM].T
    Bt = W_q[:, DIM:].T
    src = edge_index[0].astype(jnp.int32)
    dst = edge_index[1].astype(jnp.int32)
    lidx = last_idx.astype(jnp.int32)

    qv = _compute_qv(h_v, At)
    ec = n_edge // NCHUNK
    p_c = _compute_p_chunk(h_p, Bt, 0, ec)
    sc_first = _make_sc_chunk(n_tgt, ec, 0, True)
    partial, last_feat = sc_first(qv, p_c, h_t, src, dst, lidx, h_v)
    for c in range(1, NCHUNK):
        p_c = _compute_p_chunk(h_p, Bt, c, ec)
        sc_rest = _make_sc_chunk(n_tgt, ec, c * ec, False)
        partial = sc_rest(qv, p_c, h_t, src, dst, partial)
    target_ft = _sum_partials(partial)
    return (target_ft, last_feat)


# speedup vs baseline: 1.1266x; 1.1266x over previous
import functools

import jax
import jax.numpy as jnp
from jax import lax
from jax.experimental import pallas as pl
from jax.experimental.pallas import tpu as pltpu
from jax.experimental.pallas import tpu_sc as plsc

DIM = 128
NC = 2
NS = 16
NW = NC * NS
EB = 32
GRP = 4
CHUNK = 12
NCHUNK = 2


def _qv_body(hv_ref, at_ref, out_ref):
    hv = hv_ref[...]
    z = jnp.dot(hv, at_ref[...], preferred_element_type=jnp.float32)
    out_ref[:, :DIM] = jnp.exp(2.0 * z)
    out_ref[:, DIM:] = hv


def _compute_qv(h_v, At):
    n = h_v.shape[0]
    blk = 400
    return pl.pallas_call(
        _qv_body,
        grid=(n // blk,),
        in_specs=[pl.BlockSpec((blk, DIM), lambda i: (i, 0)),
                  pl.BlockSpec((DIM, DIM), lambda i: (0, 0))],
        out_specs=pl.BlockSpec((blk, 2 * DIM), lambda i: (i, 0)),
        out_shape=jax.ShapeDtypeStruct((n, 2 * DIM), jnp.float32),
    )(h_v, At)


def _p_body(hp_ref, bt_ref, out_ref):
    z = jnp.dot(hp_ref[...], bt_ref[...], preferred_element_type=jnp.float32)
    out_ref[...] = jnp.exp(2.0 * z)


def _compute_p_chunk(h_p, Bt, c, ec):
    blk = 2000
    nblk = ec // blk
    return pl.pallas_call(
        _p_body,
        grid=(nblk,),
        in_specs=[pl.BlockSpec((blk, DIM), lambda i, c=c, nblk=nblk:
                               (c * nblk + i, 0)),
                  pl.BlockSpec((DIM, DIM), lambda i: (0, 0))],
        out_specs=pl.BlockSpec((blk, DIM), lambda i: (i, 0)),
        out_shape=jax.ShapeDtypeStruct((ec, DIM), jnp.float32),
    )(h_p, Bt)


def _sum_body(p_ref, o_ref):
    o_ref[...] = p_ref[0] + p_ref[1]


def _sum_partials(partials):
    n = partials.shape[1]
    blk = 1000
    return pl.pallas_call(
        _sum_body,
        grid=(n // blk,),
        in_specs=[pl.BlockSpec((2, blk, DIM), lambda i: (0, i, 0))],
        out_specs=pl.BlockSpec((blk, DIM), lambda i: (i, 0)),
        out_shape=jax.ShapeDtypeStruct((n, DIM), jnp.float32),
    )(partials)


def _make_sc_chunk(n_tgt, e_chunk, cbase, first):
    epw = e_chunk // NW
    nb = (epw // EB) - (epw // EB) % GRP
    ngrp = nb // GRP
    e_tail = epw - nb * EB
    rpt = (n_tgt // NS) - (n_tgt // NS) % 8
    r_tail = n_tgt - rpt * NS
    lpw = n_tgt // NW - (n_tgt // NW) % 8
    lf_full = lpw // EB
    lf_rem = lpw - lf_full * EB
    lf_tail = n_tgt - lpw * NW

    mesh = plsc.VectorSubcoreMesh(core_axis_name="c", subcore_axis_name="s")
    part_ty = jax.ShapeDtypeStruct((NC, n_tgt, DIM), jnp.float32)
    out_ty = ((part_ty, jax.ShapeDtypeStruct((n_tgt, DIM), jnp.float32))
              if first else part_ty)
    scratch = [
        pltpu.VMEM((CHUNK * EB,), jnp.int32),
        pltpu.VMEM((CHUNK * EB,), jnp.int32),
    ] + [pltpu.VMEM((EB,), jnp.int32)] * GRP
    scratch += [pltpu.VMEM((EB,), jnp.int32)] * GRP
    scratch += [
        pltpu.VMEM((EB, 2 * DIM), jnp.float32),
        pltpu.VMEM((EB, 2 * DIM), jnp.float32),
        pltpu.VMEM((EB, DIM), jnp.float32),
        pltpu.VMEM((EB, DIM), jnp.float32),
        pltpu.VMEM((EB, DIM), jnp.float32),
        pltpu.VMEM((EB, DIM), jnp.float32),
        pltpu.VMEM((EB, DIM), jnp.float32),
        pltpu.VMEM((EB, DIM), jnp.float32),
        pltpu.VMEM((16,), jnp.int32),
        pltpu.VMEM((16,), jnp.int32),
        pltpu.VMEM_SHARED((n_tgt, DIM), jnp.float32),
    ] + [pltpu.SemaphoreType.DMA] * 9

    def run(qv_hbm, p_hbm, ht_hbm, src_hbm, dst_hbm, part_out, scr,
            lidx_hbm, hv_hbm, last_out, part_in):
        (srcc, dstc, ss0, ss1, ss2, ss3, ds0, ds1, ds2, ds3,
         qv0, qv1, t0, t1, p0, p1, o0, o1, stail, dtail, acc,
         sq0, sq1, st0, st1, sp0, sp1, ssc0, ssc1, sem_l) = scr
        ssc = (ss0, ss1, ss2, ss3)
        dsc = (ds0, ds1, ds2, ds3)
        qvb = (qv0, qv1)
        tbufs = (t0, t1)
        pbufs = (p0, p1)
        obufs = (o0, o1)
        sem_q = (sq0, sq1)
        sem_t = (st0, st1)
        sem_p = (sp0, sp1)
        sem_s = (ssc0, ssc1)

        cid = lax.axis_index("c")
        sid = lax.axis_index("s")
        wid = cid * NS + sid

        zero16 = jnp.zeros((16,), jnp.float32)
        r0 = sid * rpt

        if part_in is None:
            def zrow(i, _):
                for g in range(8):
                    o0[i, pl.ds(g * 16, 16)] = zero16
                return 0
            lax.fori_loop(0, EB, zrow, 0)
            zfull = rpt // EB
            zrem = rpt - zfull * EB
            for k in range(zfull):
                pltpu.sync_copy(o0, acc.at[pl.ds(r0 + k * EB, EB)])
            if zrem:
                pltpu.sync_copy(o0.at[pl.ds(0, zrem)],
                                acc.at[pl.ds(r0 + zfull * EB, zrem)])
            if r_tail:
                @pl.when(sid == 0)
                def _():
                    pltpu.sync_copy(o0.at[pl.ds(0, r_tail)],
                                    acc.at[pl.ds(rpt * NS, r_tail)])
        else:
            pltpu.sync_copy(part_in.at[cid].at[pl.ds(r0, rpt)],
                            acc.at[pl.ds(r0, rpt)])
            if r_tail:
                @pl.when(sid == 0)
                def _():
                    pltpu.sync_copy(part_in.at[cid].at[pl.ds(rpt * NS, r_tail)],
                                    acc.at[pl.ds(rpt * NS, r_tail)])
        plsc.subcore_barrier()

        ibase = cbase + wid * epw
        pbase = wid * epw

        def copy_idx_to_ring(slot, off):
            for g in range(EB // 16):
                ssc[slot][pl.ds(g * 16, 16)] = srcc[pl.ds(off + g * 16, 16)]
                dsc[slot][pl.ds(g * 16, 16)] = dstc[pl.ds(off + g * 16, 16)]

        def issue_gathers(slot, par, b):
            pltpu.async_copy(qv_hbm.at[ssc[slot]], qvb[par], sem_q[par])
            pltpu.async_copy(ht_hbm.at[dsc[slot]], tbufs[par], sem_t[par])
            pltpu.async_copy(p_hbm.at[pl.ds(b, EB)], pbufs[par], sem_p[par])

        def wait_gathers(slot, par, b):
            pltpu.make_async_copy(qv_hbm.at[ssc[slot]], qvb[par],
                                  sem_q[par]).wait()
            pltpu.make_async_copy(ht_hbm.at[dsc[slot]], tbufs[par],
                                  sem_t[par]).wait()
            pltpu.make_async_copy(p_hbm.at[pl.ds(b, EB)], pbufs[par],
                                  sem_p[par]).wait()

        def wait_scatter(par, slot):
            pltpu.make_async_copy(obufs[par], acc.at[dsc[slot]],
                                  sem_s[par]).wait()

        def compute_batch(par, n_rows):
            qv = qvb[par]
            tt = tbufs[par]
            pp = pbufs[par]
            oo = obufs[par]

            @plsc.parallel_loop(0, n_rows, unroll=4)
            def edge(i):
                s_acc = zero16
                t_acc = zero16
                for g in range(8):
                    sl = pl.ds(g * 16, 16)
                    s_acc = s_acc + tt[i, sl] / (pp[i, sl] * qv[i, sl] + 1.0)
                    t_acc = t_acc + tt[i, sl]
                coef = jnp.sum(t_acc - 2.0 * s_acc)
                for g in range(8):
                    oo[i, pl.ds(g * 16, 16)] = (
                        coef * qv[i, pl.ds(DIM + g * 16, 16)])

        pltpu.sync_copy(src_hbm.at[pl.ds(ibase, CHUNK * EB)], srcc)
        pltpu.sync_copy(dst_hbm.at[pl.ds(ibase, CHUNK * EB)], dstc)
        for b0 in (0, 1):
            copy_idx_to_ring(b0, b0 * EB)
            issue_gathers(b0, b0, pbase + b0 * EB)

        def group(g, _):
            for j in range(GRP):
                par = j % 2
                k = GRP * g + j
                wait_gathers(j, par, pbase + k * EB)
                if j < 2:
                    @pl.when(g > 0)
                    def _():
                        wait_scatter(par, (j + 2) % GRP)
                else:
                    wait_scatter(par, (j + 2) % GRP)
                compute_batch(par, EB)
                pltpu.async_copy(obufs[par], acc.at[dsc[j]], sem_s[par],
                                 add=True)
                if j == 2:
                    @pl.when((g % (CHUNK // GRP) == (CHUNK // GRP) - 1)
                             & (g < ngrp - 1))
                    def _():
                        cb = ibase + (k + 2) * EB
                        pltpu.sync_copy(src_hbm.at[pl.ds(cb, CHUNK * EB)],
                                        srcc)
                        pltpu.sync_copy(dst_hbm.at[pl.ds(cb, CHUNK * EB)],
                                        dstc)
                nslot = (j + 2) % GRP

                def prefetch():
                    k2 = GRP * g + j + 2
                    off = (k2 % CHUNK) * EB
                    copy_idx_to_ring(nslot, off)
                    issue_gathers(nslot, par, pbase + k2 * EB)
                if j < 2:
                    prefetch()
                else:
                    @pl.when(g < ngrp - 1)
                    def _():
                        prefetch()
            return 0
        lax.fori_loop(0, ngrp, group, 0)
        wait_scatter(0, (nb - 2) % GRP)
        wait_scatter(1, (nb - 1) % GRP)

        if e_tail:
            izero = jnp.zeros((16,), jnp.int32)
            stail[pl.ds(0, 16)] = izero
            dtail[pl.ds(0, 16)] = izero
            pltpu.sync_copy(src_hbm.at[pl.ds(ibase + nb * EB, e_tail)],
                            stail.at[pl.ds(0, e_tail)])
            pltpu.sync_copy(dst_hbm.at[pl.ds(ibase + nb * EB, e_tail)],
                            dtail.at[pl.ds(0, e_tail)])
            pltpu.async_copy(qv_hbm.at[stail], qv0.at[pl.ds(0, 16)],
                             sq0).wait()
            pltpu.async_copy(ht_hbm.at[dtail], t0.at[pl.ds(0, 16)],
                             st0).wait()
            pltpu.sync_copy(p_hbm.at[pl.ds(pbase + nb * EB, e_tail)],
                            p0.at[pl.ds(0, e_tail)])
            compute_batch(0, 16)
            for i in range(e_tail, 16):
                for g in range(8):
                    o0[i, pl.ds(g * 16, 16)] = zero16
            pltpu.sync_copy(o0.at[pl.ds(0, 16)], acc.at[dtail], add=True)

        if first:
            lbase = wid * lpw
            for r in range(lf_full):
                pltpu.sync_copy(lidx_hbm.at[pl.ds(lbase + r * EB, EB)], ss0)
                pltpu.async_copy(hv_hbm.at[ss0], p0, sem_l).wait()
                pltpu.sync_copy(p0, last_out.at[pl.ds(lbase + r * EB, EB)])
            if lf_rem:
                rb = lbase + lf_full * EB
                pltpu.sync_copy(lidx_hbm.at[pl.ds(rb, lf_rem)],
                                ss0.at[pl.ds(0, lf_rem)])
                pltpu.async_copy(hv_hbm.at[ss0.at[pl.ds(0, lf_rem)]],
                                 p0.at[pl.ds(0, lf_rem)], sem_l).wait()
                pltpu.sync_copy(p0.at[pl.ds(0, lf_rem)],
                                last_out.at[pl.ds(rb, lf_rem)])
            if lf_tail:
                @pl.when(wid == NW - 1)
                def _():
                    tb = lpw * NW
                    pltpu.sync_copy(lidx_hbm.at[pl.ds(tb, lf_tail)], stail)
                    pltpu.async_copy(hv_hbm.at[stail],
                                     p0.at[pl.ds(0, lf_tail)], sem_l).wait()
                    pltpu.sync_copy(p0.at[pl.ds(0, lf_tail)],
                                    last_out.at[pl.ds(tb, lf_tail)])

        plsc.subcore_barrier()
        pltpu.sync_copy(acc.at[pl.ds(r0, rpt)],
                        part_out.at[cid].at[pl.ds(r0, rpt)])
        if r_tail:
            @pl.when(sid == 0)
            def _():
                pltpu.sync_copy(acc.at[pl.ds(rpt * NS, r_tail)],
                                part_out.at[cid].at[pl.ds(rpt * NS, r_tail)])

    kw = dict(out_type=out_ty, mesh=mesh,
              compiler_params=pltpu.CompilerParams(needs_layout_passes=False),
              scratch_types=scratch)
    if first:
        @functools.partial(pl.kernel, **kw)
        def sc_first(qv_hbm, p_hbm, ht_hbm, src_hbm, dst_hbm, lidx_hbm,
                     hv_hbm, part_out, last_out,
                     srcc, dstc, ss0, ss1, ss2, ss3, ds0, ds1, ds2, ds3,
                     qv0, qv1, t0, t1, p0, p1, o0, o1, stail, dtail, acc,
                     sq0, sq1, st0, st1, sp0, sp1, ssc0, ssc1, sem_l):
            scr = (srcc, dstc, ss0, ss1, ss2, ss3, ds0, ds1, ds2, ds3,
                   qv0, qv1, t0, t1, p0, p1, o0, o1, stail, dtail, acc,
                   sq0, sq1, st0, st1, sp0, sp1, ssc0, ssc1, sem_l)
            run(qv_hbm, p_hbm, ht_hbm, src_hbm, dst_hbm, part_out, scr,
                lidx_hbm, hv_hbm, last_out, None)
        return sc_first
    else:
        @functools.partial(pl.kernel, **kw)
        def sc_rest(qv_hbm, p_hbm, ht_hbm, src_hbm, dst_hbm, part_in,
                    part_out,
                    srcc, dstc, ss0, ss1, ss2, ss3, ds0, ds1, ds2, ds3,
                    qv0, qv1, t0, t1, p0, p1, o0, o1, stail, dtail, acc,
                    sq0, sq1, st0, st1, sp0, sp1, ssc0, ssc1, sem_l):
            scr = (srcc, dstc, ss0, ss1, ss2, ss3, ds0, ds1, ds2, ds3,
                   qv0, qv1, t0, t1, p0, p1, o0, o1, stail, dtail, acc,
                   sq0, sq1, st0, st1, sp0, sp1, ssc0, ssc1, sem_l)
            run(qv_hbm, p_hbm, ht_hbm, src_hbm, dst_hbm, part_out, scr,
                None, None, None, part_in)
        return sc_rest


def kernel(h_v, h_p, h_t, edge_index, last_idx, W_q):
    n_tgt = h_t.shape[0]
    n_edge = h_p.shape[0]
    At = W_q[:, :DIM].T
    Bt = W_q[:, DIM:].T
    src = edge_index[0].astype(jnp.int32)
    dst = edge_index[1].astype(jnp.int32)
    lidx = last_idx.astype(jnp.int32)

    qv = _compute_qv(h_v, At)
    ec = n_edge // NCHUNK
    p_c = _compute_p_chunk(h_p, Bt, 0, ec)
    sc_first = _make_sc_chunk(n_tgt, ec, 0, True)
    partial, last_feat = sc_first(qv, p_c, h_t, src, dst, lidx, h_v)
    for c in range(1, NCHUNK):
        p_c = _compute_p_chunk(h_p, Bt, c, ec)
        sc_rest = _make_sc_chunk(n_tgt, ec, c * ec, False)
        partial = sc_rest(qv, p_c, h_t, src, dst, partial)
    target_ft = _sum_partials(partial)
    return (target_ft, last_feat)
